# baseline (speedup 1.0000x reference)
import functools

import jax
import jax.numpy as jnp
from jax import lax
from jax.experimental import pallas as pl
from jax.experimental.pallas import tpu as pltpu
from jax.experimental.pallas import tpu_sc as plsc

VOCAB = 1000000
MAX_LEN = 512
DIM = 64
B = 1024
S = 512

NC = 2
NS = 16
NW = NC * NS
ROWS = B * S
RPW = ROWS // NW
C = 128
NCHUNK = RPW // C
NBUF = 4
TBUF = 2
NDT = DIM // 8


def _body(idx_hbm, seg_hbm, tok_hbm, pos_hbm, segtab_hbm, out_hbm,
          base_v, segtab_v, idx_v, sgv_v, buf_v, t_v,
          gsem, ssem, isem, msem):
    cid = lax.axis_index("c")
    sid = lax.axis_index("s")
    wid = sid * NC + cid

    pltpu.sync_copy(pos_hbm, base_v.at[pl.ds(0, S), :])
    pltpu.sync_copy(pos_hbm, base_v.at[pl.ds(S, S), :])
    pltpu.sync_copy(segtab_hbm, segtab_v)

    seg_rows = [[segtab_v[k, pl.ds(j * 16, 16)] for j in range(4)]
                for k in range(2)]

    def build(r, carry):
        for j in range(4):
            sl = pl.ds(j * 16, 16)
            plsc.addupdate(base_v.at[r, sl], seg_rows[0][j])
            plsc.addupdate(base_v.at[S + r, sl], seg_rows[1][j])
        return carry

    lax.fori_loop(0, S, build, 0)

    row0 = wid * RPW
    lanes = lax.iota(jnp.int32, 16)

    def idx_copies(c, b):
        base = row0 + c * C
        return (
            pltpu.make_async_copy(idx_hbm.at[pl.ds(base, C)], idx_v.at[b],
                                  isem.at[b]),
            pltpu.make_async_copy(seg_hbm.at[pl.ds(base, C)], sgv_v.at[b],
                                  msem.at[b]),
        )

    def gather_copy(b):
        return pltpu.make_async_copy(tok_hbm.at[idx_v.at[b]], buf_v.at[b],
                                     gsem.at[b])

    def scatter_copies(c, tb):
        base = row0 + c * C
        bb = base // S
        stt = (base % S) // C
        return [
            pltpu.make_async_copy(t_v.at[tb, pl.ds(dt * 8, 8), :],
                                  out_hbm.at[bb, dt, stt, :, :],
                                  ssem.at[tb])
            for dt in range(NDT)
        ]

    for b in range(2):
        for cp in idx_copies(b, b):
            cp.start()
    for cp in idx_copies(0, 0):
        cp.wait()
    gather_copy(0).start()

    def outer(t, carry):
        for b in range(NBUF):
            c = t * NBUF + b
            tb = b % TBUF
            gather_copy(b).wait()
            @pl.when(c + 2 < NCHUNK)
            def _():
                for cp in idx_copies(c + 2, (c + 2) % NBUF):
                    cp.start()
            bn = (b + 1) % NBUF

            @pl.when(c + 1 < NCHUNK)
            def _():
                for cp in idx_copies(c + 1, bn):
                    cp.wait()
                gather_copy(bn).start()

            @pl.when(c >= TBUF)
            def _():
                for cp in scatter_copies(c - TBUF, tb):
                    cp.wait()

            m0 = lax.rem(c * C, S)
            bsplat = jnp.full((16,), b, jnp.int32)

            def group(sg, gcarry):
                svec = sg * 16 + lanes
                sgvec = sgv_v[b, pl.ds(sg * 16, 16)]
                brows = sgvec * S + (m0 + sg * 16) + lanes
                for d in range(DIM):
                    dv = jnp.full((16,), d, jnp.int32)
                    tok = plsc.load_gather(buf_v, [bsplat, svec, dv])
                    bas = plsc.load_gather(base_v, [brows, dv])
                    t_v[tb, d, pl.ds(sg * 16, 16)] = tok + bas
                return gcarry

            lax.fori_loop(0, C // 16, group, 0)
            for cp in scatter_copies(c, tb):
                cp.start()
        return carry

    lax.fori_loop(0, NCHUNK // NBUF, outer, 0)

    for k in range(TBUF):
        c = NCHUNK - TBUF + k
        for cp in scatter_copies(c, c % TBUF):
            cp.wait()


@jax.jit
def _run(idx2, seg_flat, tok2, pos_table, seg_table):
    mesh = plsc.VectorSubcoreMesh(core_axis_name="c", subcore_axis_name="s")
    f = functools.partial(
        pl.kernel,
        out_type=jax.ShapeDtypeStruct((B, NDT, S // C, 8, C), jnp.float32),
        mesh=mesh,
        scratch_types=[
            pltpu.VMEM((2 * S, DIM), jnp.float32),
            pltpu.VMEM((2, DIM), jnp.float32),
            pltpu.VMEM((NBUF, C), jnp.int32),
            pltpu.VMEM((NBUF, C), jnp.int32),
            pltpu.VMEM((NBUF, C, DIM), jnp.float32),
            pltpu.VMEM((TBUF, DIM, C), jnp.float32),
            pltpu.SemaphoreType.DMA((NBUF,)),
            pltpu.SemaphoreType.DMA((TBUF,)),
            pltpu.SemaphoreType.DMA((NBUF,)),
            pltpu.SemaphoreType.DMA((NBUF,)),
        ],
        compiler_params=pltpu.CompilerParams(use_tc_tiling_on_sc=False,
                                             needs_layout_passes=False),
    )(_body)
    return f(idx2, seg_flat, tok2, pos_table, seg_table)


NI = 4096


def _repack_body(x_ref, o_ref):
    x = x_ref[...]
    o_ref[...] = jnp.concatenate(
        [x.T, jnp.zeros((NI, DIM), jnp.float32)], axis=1)


@jax.jit
def _repack(tok_t):
    return pl.pallas_call(
        _repack_body,
        grid=((VOCAB + NI - 1) // NI,),
        in_specs=[pl.BlockSpec((DIM, NI), lambda i: (0, i))],
        out_specs=pl.BlockSpec((NI, 2 * DIM), lambda i: (i, 0)),
        out_shape=jax.ShapeDtypeStruct((VOCAB, 2 * DIM), jnp.float32),
    )(tok_t)


def kernel(sequences, segments, token_table, pos_table, seg_table):
    seq_flat = sequences.reshape(ROWS).astype(jnp.int32)
    seg_flat = segments.reshape(ROWS).astype(jnp.int32)
    idx2 = seq_flat * 2
    tok_pad = _repack(token_table.T)
    tok2 = tok_pad.reshape(2 * VOCAB, DIM)
    out5 = _run(idx2, seg_flat, tok2, pos_table, seg_table)
    return jnp.transpose(out5, (0, 2, 4, 1, 3)).reshape(B, S, DIM)


# speedup vs baseline: 2.2047x; 2.2047x over previous
import functools

import jax
import jax.numpy as jnp
from jax import lax
from jax.experimental import pallas as pl
from jax.experimental.pallas import tpu as pltpu
from jax.experimental.pallas import tpu_sc as plsc

VOCAB = 1000000
MAX_LEN = 512
DIM = 64
B = 1024
S = 512

NC = 2
NS = 16
NW = NC * NS
ROWS = B * S
RPW = ROWS // NW
C = 128
NCHUNK = RPW // C
NBUF = 4


def _body(idx_hbm, seg_hbm, tok_hbm, pos_hbm, segtab_hbm, out_hbm,
          base_v, segtab_v, idx_v, sgv_v, buf_v,
          gsem, ssem, isem, msem):
    cid = lax.axis_index("c")
    sid = lax.axis_index("s")
    wid = sid * NC + cid

    pltpu.sync_copy(pos_hbm, base_v.at[pl.ds(0, S), :])
    pltpu.sync_copy(pos_hbm, base_v.at[pl.ds(S, S), :])
    pltpu.sync_copy(segtab_hbm, segtab_v)

    seg_rows = [[segtab_v[k, pl.ds(j * 16, 16)] for j in range(4)]
                for k in range(2)]

    def build(r, carry):
        for j in range(4):
            sl = pl.ds(j * 16, 16)
            plsc.addupdate(base_v.at[r, sl], seg_rows[0][j])
            plsc.addupdate(base_v.at[S + r, sl], seg_rows[1][j])
        return carry

    lax.fori_loop(0, S, build, 0)

    row0 = wid * RPW
    lanes = lax.iota(jnp.int32, 16)

    def idx_copies(c, b):
        base = row0 + c * C
        return (
            pltpu.make_async_copy(idx_hbm.at[pl.ds(base, C)], idx_v.at[b],
                                  isem.at[b]),
            pltpu.make_async_copy(seg_hbm.at[pl.ds(base, C)], sgv_v.at[b],
                                  msem.at[b]),
        )

    def gather_copy(b):
        return pltpu.make_async_copy(tok_hbm.at[idx_v.at[b]], buf_v.at[b],
                                     gsem.at[b])

    def scatter_copy(c, b):
        base = row0 + c * C
        return pltpu.make_async_copy(buf_v.at[b],
                                     out_hbm.at[pl.ds(base, C), :],
                                     ssem.at[b])

    for b in range(2):
        for cp in idx_copies(b, b):
            cp.start()
    for cp in idx_copies(0, 0):
        cp.wait()
    gather_copy(0).start()

    def outer(t, carry):
        for b in range(NBUF):
            c = t * NBUF + b
            gather_copy(b).wait()
            @pl.when(c + 2 < NCHUNK)
            def _():
                for cp in idx_copies(c + 2, (c + 2) % NBUF):
                    cp.start()
            bn = (b + 1) % NBUF

            @pl.when(c + 1 < NCHUNK)
            def _():
                for cp in idx_copies(c + 1, bn):
                    cp.wait()

                @pl.when(c >= 3)
                def _():
                    scatter_copy(c - 3, bn).wait()

                gather_copy(bn).start()

            m0 = lax.rem(c * C, S)

            def group(g, gcarry):
                sgvec = sgv_v[b, pl.ds(g * 16, 16)]
                brows = sgvec * S + (m0 + g * 16) + lanes
                for r in range(16):
                    i = g * 16 + r
                    br = brows[r]
                    for j in range(4):
                        sl = pl.ds(j * 16, 16)
                        plsc.addupdate(buf_v.at[b, i, sl], base_v[br, sl])
                return gcarry

            lax.fori_loop(0, C // 16, group, 0)
            scatter_copy(c, b).start()
        return carry

    lax.fori_loop(0, NCHUNK // NBUF, outer, 0)

    for b in range(NBUF):
        c = NCHUNK - NBUF + b
        scatter_copy(c, b).wait()


@jax.jit
def _run(idx2, seg_flat, tok2, pos_table, seg_table):
    mesh = plsc.VectorSubcoreMesh(core_axis_name="c", subcore_axis_name="s")
    f = functools.partial(
        pl.kernel,
        out_type=jax.ShapeDtypeStruct((ROWS, DIM), jnp.float32),
        mesh=mesh,
        scratch_types=[
            pltpu.VMEM((2 * S, DIM), jnp.float32),
            pltpu.VMEM((2, DIM), jnp.float32),
            pltpu.VMEM((NBUF, C), jnp.int32),
            pltpu.VMEM((NBUF, C), jnp.int32),
            pltpu.VMEM((NBUF, C, DIM), jnp.float32),
            pltpu.SemaphoreType.DMA((NBUF,)),
            pltpu.SemaphoreType.DMA((NBUF,)),
            pltpu.SemaphoreType.DMA((NBUF,)),
            pltpu.SemaphoreType.DMA((NBUF,)),
        ],
        compiler_params=pltpu.CompilerParams(use_tc_tiling_on_sc=False),
    )(_body)
    return f(idx2, seg_flat, tok2, pos_table, seg_table)


NI = 8192


def _repack_body(x_ref, o_ref):
    x = x_ref[...]
    o_ref[...] = jnp.concatenate(
        [x.T, jnp.zeros((NI, DIM), jnp.float32)], axis=1)


@jax.jit
def _repack(tok_t):
    return pl.pallas_call(
        _repack_body,
        grid=((VOCAB + NI - 1) // NI,),
        in_specs=[pl.BlockSpec((DIM, NI), lambda i: (0, i))],
        out_specs=pl.BlockSpec((NI, 2 * DIM), lambda i: (i, 0)),
        out_shape=jax.ShapeDtypeStruct((VOCAB, 2 * DIM), jnp.float32),
    )(tok_t)


def kernel(sequences, segments, token_table, pos_table, seg_table):
    seq_flat = sequences.reshape(ROWS).astype(jnp.int32)
    seg_flat = segments.reshape(ROWS).astype(jnp.int32)
    idx2 = seq_flat * 2
    tok_pad = _repack(token_table.T)
    tok2 = tok_pad.reshape(2 * VOCAB, DIM)
    out = _run(idx2, seg_flat, tok2, pos_table, seg_table)
    return out.reshape(B, S, DIM)


# speedup vs baseline: 2.2538x; 1.0223x over previous
import functools

import jax
import jax.numpy as jnp
from jax import lax
from jax.experimental import pallas as pl
from jax.experimental.pallas import tpu as pltpu
from jax.experimental.pallas import tpu_sc as plsc

VOCAB = 1000000
MAX_LEN = 512
DIM = 64
B = 1024
S = 512

NC = 2
NS = 16
NW = NC * NS
ROWS = B * S
RPW = ROWS // NW
C = 128
NCHUNK = RPW // C
NBUF = 4


def _body(idx_hbm, seg_hbm, tok_hbm, pos_hbm, segtab_hbm, out_hbm,
          base_v, segtab_v, idx_v, sgv_v, buf_v,
          gsem, ssem, isem, msem):
    cid = lax.axis_index("c")
    sid = lax.axis_index("s")
    wid = sid * NC + cid

    pltpu.sync_copy(pos_hbm, base_v.at[pl.ds(0, S), :])
    pltpu.sync_copy(pos_hbm, base_v.at[pl.ds(S, S), :])
    pltpu.sync_copy(segtab_hbm, segtab_v)

    seg_rows = [[segtab_v[k, pl.ds(j * 16, 16)] for j in range(4)]
                for k in range(2)]

    def build(r, carry):
        for j in range(4):
            sl = pl.ds(j * 16, 16)
            plsc.addupdate(base_v.at[r, sl], seg_rows[0][j])
            plsc.addupdate(base_v.at[S + r, sl], seg_rows[1][j])
        return carry

    lax.fori_loop(0, S, build, 0)

    row0 = wid * RPW
    lanes = lax.iota(jnp.int32, 16)

    def idx_copies(c, b):
        base = row0 + c * C
        return (
            pltpu.make_async_copy(idx_hbm.at[pl.ds(base, C)], idx_v.at[b],
                                  isem.at[b]),
            pltpu.make_async_copy(seg_hbm.at[pl.ds(base, C)], sgv_v.at[b],
                                  msem.at[b]),
        )

    def gather_copy(b):
        return pltpu.make_async_copy(tok_hbm.at[idx_v.at[b]], buf_v.at[b],
                                     gsem.at[b])

    def scatter_copy(c, b):
        base = row0 + c * C
        return pltpu.make_async_copy(buf_v.at[b],
                                     out_hbm.at[pl.ds(base, C), :],
                                     ssem.at[b])

    for b in range(2):
        for cp in idx_copies(b, b):
            cp.start()
    for cp in idx_copies(0, 0):
        cp.wait()
    gather_copy(0).start()

    def outer(t, carry):
        for b in range(NBUF):
            c = t * NBUF + b
            gather_copy(b).wait()
            @pl.when(c + 2 < NCHUNK)
            def _():
                for cp in idx_copies(c + 2, (c + 2) % NBUF):
                    cp.start()
            bn = (b + 1) % NBUF

            @pl.when(c + 1 < NCHUNK)
            def _():
                for cp in idx_copies(c + 1, bn):
                    cp.wait()

                @pl.when(c >= 3)
                def _():
                    scatter_copy(c - 3, bn).wait()

                gather_copy(bn).start()

            m0 = lax.rem(c * C, S)

            def group(g, gcarry):
                sgvec = sgv_v[b, pl.ds(g * 16, 16)]
                brows = sgvec * S + (m0 + g * 16) + lanes
                for r in range(16):
                    i = g * 16 + r
                    br = brows[r]
                    for j in range(4):
                        sl = pl.ds(j * 16, 16)
                        plsc.addupdate(buf_v.at[b, i, sl], base_v[br, sl])
                return gcarry

            lax.fori_loop(0, C // 16, group, 0)
            scatter_copy(c, b).start()
        return carry

    lax.fori_loop(0, NCHUNK // NBUF, outer, 0)

    for b in range(NBUF):
        c = NCHUNK - NBUF + b
        scatter_copy(c, b).wait()


@jax.jit
def _run(idx2, seg_flat, tok2, pos_table, seg_table):
    mesh = plsc.VectorSubcoreMesh(core_axis_name="c", subcore_axis_name="s")
    f = functools.partial(
        pl.kernel,
        out_type=jax.ShapeDtypeStruct((ROWS, DIM), jnp.float32),
        mesh=mesh,
        scratch_types=[
            pltpu.VMEM((2 * S, DIM), jnp.float32),
            pltpu.VMEM((2, DIM), jnp.float32),
            pltpu.VMEM((NBUF, C), jnp.int32),
            pltpu.VMEM((NBUF, C), jnp.int32),
            pltpu.VMEM((NBUF, C, DIM), jnp.float32),
            pltpu.SemaphoreType.DMA((NBUF,)),
            pltpu.SemaphoreType.DMA((NBUF,)),
            pltpu.SemaphoreType.DMA((NBUF,)),
            pltpu.SemaphoreType.DMA((NBUF,)),
        ],
        compiler_params=pltpu.CompilerParams(use_tc_tiling_on_sc=False),
    )(_body)
    return f(idx2, seg_flat, tok2, pos_table, seg_table)


NI = 16384


def _repack_body(x_ref, o_ref):
    x = x_ref[...]
    o_ref[...] = jnp.concatenate(
        [x.T, jnp.zeros((NI, DIM), jnp.float32)], axis=1)


@jax.jit
def _repack(tok_t):
    return pl.pallas_call(
        _repack_body,
        grid=((VOCAB + NI - 1) // NI,),
        in_specs=[pl.BlockSpec((DIM, NI), lambda i: (0, i))],
        out_specs=pl.BlockSpec((NI, 2 * DIM), lambda i: (i, 0)),
        out_shape=jax.ShapeDtypeStruct((VOCAB, 2 * DIM), jnp.float32),
    )(tok_t)


def kernel(sequences, segments, token_table, pos_table, seg_table):
    seq_flat = sequences.reshape(ROWS).astype(jnp.int32)
    seg_flat = segments.reshape(ROWS).astype(jnp.int32)
    idx2 = seq_flat * 2
    tok_pad = _repack(token_table.T)
    tok2 = tok_pad.reshape(2 * VOCAB, DIM)
    out = _run(idx2, seg_flat, tok2, pos_table, seg_table)
    return out.reshape(B, S, DIM)


# speedup vs baseline: 2.2723x; 1.0082x over previous
import functools

import jax
import jax.numpy as jnp
from jax import lax
from jax.experimental import pallas as pl
from jax.experimental.pallas import tpu as pltpu
from jax.experimental.pallas import tpu_sc as plsc

VOCAB = 1000000
MAX_LEN = 512
DIM = 64
B = 1024
S = 512

NC = 2
NS = 16
NW = NC * NS
ROWS = B * S
RPW = ROWS // NW
C = 128
NCHUNK = RPW // C
NBUF = 4


def _body(idx_hbm, seg_hbm, tok_hbm, pos_hbm, segtab_hbm, out_hbm,
          base_v, segtab_v, idx_v, sgv_v, buf_v,
          gsem, ssem, isem, msem):
    cid = lax.axis_index("c")
    sid = lax.axis_index("s")
    wid = sid * NC + cid

    pltpu.sync_copy(pos_hbm, base_v.at[pl.ds(0, S), :])
    pltpu.sync_copy(pos_hbm, base_v.at[pl.ds(S, S), :])
    pltpu.sync_copy(segtab_hbm, segtab_v)

    seg_rows = [[segtab_v[k, pl.ds(j * 16, 16)] for j in range(4)]
                for k in range(2)]

    def build(r, carry):
        for j in range(4):
            sl = pl.ds(j * 16, 16)
            plsc.addupdate(base_v.at[r, sl], seg_rows[0][j])
            plsc.addupdate(base_v.at[S + r, sl], seg_rows[1][j])
        return carry

    lax.fori_loop(0, S, build, 0)

    row0 = wid * RPW
    lanes = lax.iota(jnp.int32, 16)

    def idx_copies(c, b):
        base = row0 + c * C
        return (
            pltpu.make_async_copy(idx_hbm.at[pl.ds(base, C)], idx_v.at[b],
                                  isem.at[b]),
            pltpu.make_async_copy(seg_hbm.at[pl.ds(base, C)], sgv_v.at[b],
                                  msem.at[b]),
        )

    def gather_copy(b):
        return pltpu.make_async_copy(tok_hbm.at[idx_v.at[b]], buf_v.at[b],
                                     gsem.at[b])

    def scatter_copy(c, b):
        base = row0 + c * C
        return pltpu.make_async_copy(buf_v.at[b],
                                     out_hbm.at[pl.ds(base, C), :],
                                     ssem.at[b])

    for b in range(2):
        for cp in idx_copies(b, b):
            cp.start()
    for cp in idx_copies(0, 0):
        cp.wait()
    gather_copy(0).start()

    def outer(t, carry):
        for b in range(NBUF):
            c = t * NBUF + b
            gather_copy(b).wait()
            @pl.when(c + 2 < NCHUNK)
            def _():
                for cp in idx_copies(c + 2, (c + 2) % NBUF):
                    cp.start()
            bn = (b + 1) % NBUF

            @pl.when(c + 1 < NCHUNK)
            def _():
                for cp in idx_copies(c + 1, bn):
                    cp.wait()

                @pl.when(c >= 3)
                def _():
                    scatter_copy(c - 3, bn).wait()

                gather_copy(bn).start()

            m0 = lax.rem(c * C, S)

            def group(g, gcarry):
                sgvec = sgv_v[b, pl.ds(g * 16, 16)]
                brows = sgvec * S + (m0 + g * 16) + lanes
                for r in range(16):
                    i = g * 16 + r
                    br = brows[r]
                    for j in range(4):
                        sl = pl.ds(j * 16, 16)
                        plsc.addupdate(buf_v.at[b, i, sl], base_v[br, sl])
                return gcarry

            lax.fori_loop(0, C // 16, group, 0)
            scatter_copy(c, b).start()
        return carry

    lax.fori_loop(0, NCHUNK // NBUF, outer, 0)

    for b in range(NBUF):
        c = NCHUNK - NBUF + b
        scatter_copy(c, b).wait()


@jax.jit
def _run(idx2, seg_flat, tok2, pos_table, seg_table):
    mesh = plsc.VectorSubcoreMesh(core_axis_name="c", subcore_axis_name="s")
    f = functools.partial(
        pl.kernel,
        out_type=jax.ShapeDtypeStruct((ROWS, DIM), jnp.float32),
        mesh=mesh,
        scratch_types=[
            pltpu.VMEM((2 * S, DIM), jnp.float32),
            pltpu.VMEM((2, DIM), jnp.float32),
            pltpu.VMEM((NBUF, C), jnp.int32),
            pltpu.VMEM((NBUF, C), jnp.int32),
            pltpu.VMEM((NBUF, C, DIM), jnp.float32),
            pltpu.SemaphoreType.DMA((NBUF,)),
            pltpu.SemaphoreType.DMA((NBUF,)),
            pltpu.SemaphoreType.DMA((NBUF,)),
            pltpu.SemaphoreType.DMA((NBUF,)),
        ],
        compiler_params=pltpu.CompilerParams(use_tc_tiling_on_sc=False),
    )(_body)
    return f(idx2, seg_flat, tok2, pos_table, seg_table)


NI = 32768


def _repack_body(x_ref, o_ref):
    x = x_ref[...]
    o_ref[...] = jnp.concatenate(
        [x.T, jnp.zeros((NI, DIM), jnp.float32)], axis=1)


@jax.jit
def _repack(tok_t):
    return pl.pallas_call(
        _repack_body,
        grid=((VOCAB + NI - 1) // NI,),
        in_specs=[pl.BlockSpec((DIM, NI), lambda i: (0, i))],
        out_specs=pl.BlockSpec((NI, 2 * DIM), lambda i: (i, 0)),
        out_shape=jax.ShapeDtypeStruct((VOCAB, 2 * DIM), jnp.float32),
    )(tok_t)


def kernel(sequences, segments, token_table, pos_table, seg_table):
    seq_flat = sequences.reshape(ROWS).astype(jnp.int32)
    seg_flat = segments.reshape(ROWS).astype(jnp.int32)
    idx2 = seq_flat * 2
    tok_pad = _repack(token_table.T)
    tok2 = tok_pad.reshape(2 * VOCAB, DIM)
    out = _run(idx2, seg_flat, tok2, pos_table, seg_table)
    return out.reshape(B, S, DIM)
